# baseline (speedup 1.0000x reference)
import functools

import jax
import jax.numpy as jnp
from jax import lax
from jax.experimental import pallas as pl
from jax.experimental.pallas import tpu as pltpu
from jax.experimental.pallas import tpu_sc as plsc

BATCH = 64
PLANE = 512 * 512
NUM_WORKERS = 32
FRAMES_PER_WORKER = BATCH // NUM_WORKERS
CHUNK = 16384
CHUNKS_PER_FRAME = PLANE // CHUNK
LANES = 16

_mesh = plsc.VectorSubcoreMesh(core_axis_name="c", subcore_axis_name="s")


@functools.partial(
    pl.kernel,
    out_type=jax.ShapeDtypeStruct((BATCH * PLANE,), jnp.float32),
    mesh=_mesh,
    scratch_types=[
        pltpu.VMEM((BATCH + 16,), jnp.float32),
        pltpu.VMEM((BATCH + 16,), jnp.float32),
        pltpu.VMEM((CHUNK,), jnp.float32),
        pltpu.VMEM((CHUNK,), jnp.float32),
        pltpu.VMEM((CHUNK,), jnp.float32),
        pltpu.VMEM((CHUNK,), jnp.float32),
        pltpu.VMEM((CHUNK,), jnp.float32),
        pltpu.VMEM((CHUNK,), jnp.float32),
        pltpu.SemaphoreType.DMA,
        pltpu.SemaphoreType.DMA,
        pltpu.SemaphoreType.DMA,
        pltpu.SemaphoreType.DMA,
    ],
)
def _blend_kernel(frames_hbm, ratio_hbm, dir_hbm, out_hbm,
                  rbuf, dbuf, b0_0, b0_1, b1_0, b1_1, ob0, ob1,
                  sin0, sin1, sout0, sout1):
    b0 = (b0_0, b0_1)
    b1 = (b1_0, b1_1)
    ob = (ob0, ob1)
    sin = (sin0, sin1)
    sout = (sout0, sout1)

    wid = lax.axis_index("s") * 2 + lax.axis_index("c")

    pltpu.sync_copy(ratio_hbm, rbuf.at[pl.ds(0, BATCH)])
    pltpu.sync_copy(dir_hbm, dbuf.at[pl.ds(0, BATCH)])

    weights = []
    for fi in range(FRAMES_PER_WORKER):
        n = wid * FRAMES_PER_WORKER + fi
        rv = rbuf[pl.ds(n, LANES)][0]
        dv = dbuf[pl.ds(n, LANES)][0]
        w1 = jnp.where(dv > 0, rv, jnp.where(dv < 0, 1.0 - rv, 0.0))
        w0 = jnp.where(dv > 0, 1.0 - rv, jnp.where(dv < 0, rv, 0.0))
        weights.append((w0, w1))

    total = FRAMES_PER_WORKER * CHUNKS_PER_FRAME

    def start_in(t, slot):
        fi, k = divmod(t, CHUNKS_PER_FRAME)
        n = wid * FRAMES_PER_WORKER + fi
        off0 = (n * 2) * PLANE + k * CHUNK
        off1 = (n * 2 + 1) * PLANE + k * CHUNK
        d0 = pltpu.async_copy(frames_hbm.at[pl.ds(off0, CHUNK)], b0[slot],
                              sin[slot])
        d1 = pltpu.async_copy(frames_hbm.at[pl.ds(off1, CHUNK)], b1[slot],
                              sin[slot])
        return d0, d1

    def start_out(t, slot):
        fi, k = divmod(t, CHUNKS_PER_FRAME)
        n = wid * FRAMES_PER_WORKER + fi
        off = n * PLANE + k * CHUNK
        return pltpu.async_copy(ob[slot], out_hbm.at[pl.ds(off, CHUNK)],
                                sout[slot])

    in_descs = {}
    out_descs = {}
    in_descs[0] = start_in(0, 0)
    for t in range(total):
        slot = t % 2
        if t + 1 < total:
            in_descs[t + 1] = start_in(t + 1, (t + 1) % 2)
        d0, d1 = in_descs.pop(t)
        d0.wait()
        d1.wait()
        if t >= 2:
            out_descs.pop(t - 2).wait()
        w0, w1 = weights[t // CHUNKS_PER_FRAME]
        src0, src1, dst = b0[slot], b1[slot], ob[slot]

        @plsc.parallel_loop(0, CHUNK // LANES, unroll=8)
        def _compute(i):
            o = i * LANES
            x0 = src0[pl.ds(o, LANES)]
            x1 = src1[pl.ds(o, LANES)]
            dst[pl.ds(o, LANES)] = w0 * x0 + w1 * x1

        out_descs[t] = start_out(t, slot)
    out_descs.pop(total - 2).wait()
    out_descs.pop(total - 1).wait()


def kernel(exist_frames, ratio, direction):
    frames_flat = exist_frames.reshape(-1)
    out_flat = _blend_kernel(frames_flat, ratio.reshape(-1),
                             direction.reshape(-1))
    return out_flat.reshape(BATCH, 1, 512, 512)


# speedup vs baseline: 2.7184x; 2.7184x over previous
import functools

import jax
import jax.numpy as jnp
from jax import lax
from jax.experimental import pallas as pl
from jax.experimental.pallas import tpu as pltpu
from jax.experimental.pallas import tpu_sc as plsc

BATCH = 64
H = 512
W = 512
NUM_WORKERS = 32
FRAMES_PER_WORKER = BATCH // NUM_WORKERS
ROWS = 32
CHUNKS_PER_FRAME = H // ROWS
LANES = 16
VECS_PER_CHUNK = ROWS * W // LANES

_mesh = plsc.VectorSubcoreMesh(core_axis_name="c", subcore_axis_name="s")


@functools.partial(
    pl.kernel,
    out_type=jax.ShapeDtypeStruct((BATCH, 1, H, W), jnp.float32),
    mesh=_mesh,
    compiler_params=pltpu.CompilerParams(use_tc_tiling_on_sc=True),
    scratch_types=[
        pltpu.VMEM((BATCH + 16,), jnp.float32),
        pltpu.VMEM((BATCH + 16,), jnp.float32),
        pltpu.VMEM((ROWS, W), jnp.float32),
        pltpu.VMEM((ROWS, W), jnp.float32),
        pltpu.VMEM((ROWS, W), jnp.float32),
        pltpu.VMEM((ROWS, W), jnp.float32),
        pltpu.VMEM((ROWS, W), jnp.float32),
        pltpu.VMEM((ROWS, W), jnp.float32),
        pltpu.SemaphoreType.DMA,
        pltpu.SemaphoreType.DMA,
        pltpu.SemaphoreType.DMA,
        pltpu.SemaphoreType.DMA,
    ],
)
def _blend_kernel(frames_hbm, ratio_hbm, dir_hbm, out_hbm,
                  rbuf, dbuf, b0_0, b0_1, b1_0, b1_1, ob0, ob1,
                  sin0, sin1, sout0, sout1):
    b0 = (b0_0, b0_1)
    b1 = (b1_0, b1_1)
    ob = (ob0, ob1)
    sin = (sin0, sin1)
    sout = (sout0, sout1)

    wid = lax.axis_index("s") * 2 + lax.axis_index("c")

    pltpu.sync_copy(ratio_hbm, rbuf.at[pl.ds(0, BATCH)])
    pltpu.sync_copy(dir_hbm, dbuf.at[pl.ds(0, BATCH)])

    weights = []
    for fi in range(FRAMES_PER_WORKER):
        n = wid * FRAMES_PER_WORKER + fi
        rv = rbuf[pl.ds(n, LANES)][0]
        dv = dbuf[pl.ds(n, LANES)][0]
        w1 = jnp.where(dv > 0, rv, jnp.where(dv < 0, 1.0 - rv, 0.0))
        w0 = jnp.where(dv > 0, 1.0 - rv, jnp.where(dv < 0, rv, 0.0))
        weights.append((w0, w1))

    total = FRAMES_PER_WORKER * CHUNKS_PER_FRAME

    def start_in(t, slot):
        fi, k = divmod(t, CHUNKS_PER_FRAME)
        n = wid * FRAMES_PER_WORKER + fi
        r0 = k * ROWS
        d0 = pltpu.async_copy(frames_hbm.at[n, 0, pl.ds(r0, ROWS), :],
                              b0[slot], sin[slot])
        d1 = pltpu.async_copy(frames_hbm.at[n, 1, pl.ds(r0, ROWS), :],
                              b1[slot], sin[slot])
        return d0, d1

    def start_out(t, slot):
        fi, k = divmod(t, CHUNKS_PER_FRAME)
        n = wid * FRAMES_PER_WORKER + fi
        r0 = k * ROWS
        return pltpu.async_copy(ob[slot], out_hbm.at[n, 0, pl.ds(r0, ROWS), :],
                                sout[slot])

    in_descs = {}
    out_descs = {}
    in_descs[0] = start_in(0, 0)
    for t in range(total):
        slot = t % 2
        if t + 1 < total:
            in_descs[t + 1] = start_in(t + 1, (t + 1) % 2)
        d0, d1 = in_descs.pop(t)
        d0.wait()
        d1.wait()
        if t >= 2:
            out_descs.pop(t - 2).wait()
        w0, w1 = weights[t // CHUNKS_PER_FRAME]
        src0, src1, dst = b0[slot], b1[slot], ob[slot]

        @plsc.parallel_loop(0, VECS_PER_CHUNK, unroll=8)
        def _compute(i):
            r = lax.shift_right_logical(i, 5)
            c = (i & 31) * LANES
            x0 = src0[r, pl.ds(c, LANES)]
            x1 = src1[r, pl.ds(c, LANES)]
            dst[r, pl.ds(c, LANES)] = w0 * x0 + w1 * x1

        out_descs[t] = start_out(t, slot)
    out_descs.pop(total - 2).wait()
    out_descs.pop(total - 1).wait()


def kernel(exist_frames, ratio, direction):
    return _blend_kernel(exist_frames, ratio.reshape(-1),
                         direction.reshape(-1))
